# Initial kernel scaffold; baseline (speedup 1.0000x reference)
import jax
import jax.numpy as jnp
from jax.experimental import pallas as pl


def kernel(x, edge_index, edge_attr, global_attr, We1, be1, We2, be2, Wn1, bn1, Wn2, bn2, Wg1, bg1, Wg2, bg2):
    raise NotImplementedError("write your pallas kernel here")


# speedup vs baseline: 3.2538x; 3.2538x over previous
import functools

import jax
import jax.numpy as jnp
from jax import lax
from jax.experimental import pallas as pl
from jax.experimental.pallas import tpu as pltpu
from jax.experimental.pallas import tpu_sc as plsc

N_NODES = 10000
N_EDGES = 320000
DN = 128
DE = 16
DG = 16

NC = 2
NS = 16
NW = NC * NS
EPT = N_EDGES // NW
CHUNK = 80
NCHUNK = EPT // CHUNK
N_PAD = 10240
HALF = N_PAD // 2

F32 = jnp.float32


def _dot(a, b):
    return jnp.dot(a, b, preferred_element_type=F32)


def _ab_body(x_ref, w1s_ref, w1d_ref, w1u_ref, be1_ref, u_ref, a_ref, b_ref):
    ub = _dot(u_ref[...], w1u_ref[...]) + be1_ref[...]
    a_ref[...] = _dot(x_ref[...], w1s_ref[...]) + ub
    b_ref[...] = _dot(x_ref[...], w1d_ref[...])


def _make_tables(x, w1s, w1d, w1u, be1, u):
    blk = 2000
    grid = (N_NODES // blk,)
    return pl.pallas_call(
        _ab_body,
        grid=grid,
        in_specs=[
            pl.BlockSpec((blk, DN), lambda i: (i, 0)),
            pl.BlockSpec((DN, DN), lambda i: (0, 0)),
            pl.BlockSpec((DN, DN), lambda i: (0, 0)),
            pl.BlockSpec((DG, DN), lambda i: (0, 0)),
            pl.BlockSpec((1, DN), lambda i: (0, 0)),
            pl.BlockSpec((1, DG), lambda i: (0, 0)),
        ],
        out_specs=[
            pl.BlockSpec((blk, DN), lambda i: (i, 0)),
            pl.BlockSpec((blk, DN), lambda i: (i, 0)),
        ],
        out_shape=[
            jax.ShapeDtypeStruct((N_NODES, DN), F32),
            jax.ShapeDtypeStruct((N_NODES, DN), F32),
        ],
    )(x, w1s, w1d, w1u, be1, u)


def _gather_body(a_hbm, b_hbm, row_hbm, col_hbm, ga_hbm, gb_hbm,
                 ridx, cidx, ra, rb, sa, sb):
    cid = lax.axis_index("c")
    sid = lax.axis_index("s")
    wid = sid * NC + cid
    pltpu.sync_copy(row_hbm.at[wid], ridx)
    pltpu.sync_copy(col_hbm.at[wid], cidx)

    def body(g, carry):
        base = wid * EPT + g * CHUNK
        ca = pltpu.async_copy(a_hbm.at[ridx.at[g]], ra, sa)
        cb = pltpu.async_copy(b_hbm.at[cidx.at[g]], rb, sb)
        ca.wait()
        cb.wait()
        pltpu.sync_copy(ra, ga_hbm.at[pl.ds(base, CHUNK)])
        pltpu.sync_copy(rb, gb_hbm.at[pl.ds(base, CHUNK)])
        return carry

    lax.fori_loop(0, NCHUNK, body, 0)


def _gather_rows(a_tab, b_tab, row3, col3):
    mesh = plsc.VectorSubcoreMesh(core_axis_name="c", subcore_axis_name="s")
    k = functools.partial(
        pl.kernel,
        out_type=(
            jax.ShapeDtypeStruct((N_EDGES, DN), F32),
            jax.ShapeDtypeStruct((N_EDGES, DN), F32),
        ),
        mesh=mesh,
        scratch_types=[
            pltpu.VMEM((NCHUNK, CHUNK), jnp.int32),
            pltpu.VMEM((NCHUNK, CHUNK), jnp.int32),
            pltpu.VMEM((CHUNK, DN), F32),
            pltpu.VMEM((CHUNK, DN), F32),
            pltpu.SemaphoreType.DMA,
            pltpu.SemaphoreType.DMA,
        ],
    )(_gather_body)
    return k(a_tab, b_tab, row3, col3)


def _edge_body(ga_ref, gb_ref, ea_ref, w1e_ref, w2_ref, be2_ref, eo_ref):
    h = ga_ref[...] + gb_ref[...] + _dot(ea_ref[...], w1e_ref[...])
    h = jnp.maximum(h, 0.0)
    eo_ref[...] = _dot(h, w2_ref[...]) + be2_ref[...]


def _edge_mlp(ga, gb, edge_attr, w1e, w2, be2):
    blk = 4000
    grid = (N_EDGES // blk,)
    return pl.pallas_call(
        _edge_body,
        grid=grid,
        in_specs=[
            pl.BlockSpec((blk, DN), lambda i: (i, 0)),
            pl.BlockSpec((blk, DN), lambda i: (i, 0)),
            pl.BlockSpec((blk, DE), lambda i: (i, 0)),
            pl.BlockSpec((DE, DN), lambda i: (0, 0)),
            pl.BlockSpec((DN, DE), lambda i: (0, 0)),
            pl.BlockSpec((1, DE), lambda i: (0, 0)),
        ],
        out_specs=pl.BlockSpec((blk, DE), lambda i: (i, 0)),
        out_shape=jax.ShapeDtypeStruct((N_EDGES, DE), F32),
    )(ga, gb, edge_attr, w1e, w2, be2)


def _scatter_body(eo_hbm, col_hbm, zeros_hbm, out_hbm, cidx, buf, acc):
    cid = lax.axis_index("c")
    sid = lax.axis_index("s")
    wid = sid * NC + cid

    @pl.when(sid == 0)
    def _():
        pltpu.sync_copy(zeros_hbm, acc)

    plsc.subcore_barrier()

    def chunk(g, carry):
        pltpu.sync_copy(col_hbm.at[wid, g], cidx)
        pltpu.sync_copy(eo_hbm.at[wid, g], buf)
        pltpu.sync_copy(buf, acc.at[cidx], add=True)
        return carry

    lax.fori_loop(0, NCHUNK, chunk, 0)
    plsc.subcore_barrier()

    @pl.when(sid == 0)
    def _():
        pltpu.sync_copy(acc, out_hbm.at[cid])


def _segment_sum(eo4, col3, zeros2d):
    mesh = plsc.VectorSubcoreMesh(core_axis_name="c", subcore_axis_name="s")
    k = functools.partial(
        pl.kernel,
        out_type=jax.ShapeDtypeStruct((NC, N_PAD, DE), F32),
        mesh=mesh,
        scratch_types=[
            pltpu.VMEM((CHUNK,), jnp.int32),
            pltpu.VMEM((CHUNK, DE), F32),
            pltpu.VMEM_SHARED((N_PAD, DE), F32),
        ],
        compiler_params=pltpu.CompilerParams(use_tc_tiling_on_sc=False),
    )(_scatter_body)
    return k(eo4, col3, zeros2d)


def _node_body(x_ref, p_ref, wnx_ref, wna_ref, wnu_ref, bn1_ref, u_ref,
               wn2_ref, bn2_ref, xo_ref, cs_ref):
    i = pl.program_id(0)
    agg = jnp.sum(p_ref[...], axis=0)
    ub = _dot(u_ref[...], wnu_ref[...]) + bn1_ref[...]
    h = _dot(x_ref[...], wnx_ref[...]) + _dot(agg, wna_ref[...]) + ub
    h = jnp.maximum(h, 0.0)
    xo = _dot(h, wn2_ref[...]) + bn2_ref[...]
    xo_ref[...] = xo

    @pl.when(i == 0)
    def _():
        cs_ref[...] = jnp.zeros_like(cs_ref)

    cs_ref[...] += jnp.sum(xo, axis=0, keepdims=True)


def _node_mlp(x, partials, wnx, wna, wnu, bn1, u, wn2, bn2):
    blk = 2000
    grid = (N_NODES // blk,)
    return pl.pallas_call(
        _node_body,
        grid=grid,
        in_specs=[
            pl.BlockSpec((blk, DN), lambda i: (i, 0)),
            pl.BlockSpec((NC, blk, DE), lambda i: (0, i, 0)),
            pl.BlockSpec((DN, DN), lambda i: (0, 0)),
            pl.BlockSpec((DE, DN), lambda i: (0, 0)),
            pl.BlockSpec((DG, DN), lambda i: (0, 0)),
            pl.BlockSpec((1, DN), lambda i: (0, 0)),
            pl.BlockSpec((1, DG), lambda i: (0, 0)),
            pl.BlockSpec((DN, DN), lambda i: (0, 0)),
            pl.BlockSpec((1, DN), lambda i: (0, 0)),
        ],
        out_specs=[
            pl.BlockSpec((blk, DN), lambda i: (i, 0)),
            pl.BlockSpec((1, DN), lambda i: (0, 0)),
        ],
        out_shape=[
            jax.ShapeDtypeStruct((N_NODES, DN), F32),
            jax.ShapeDtypeStruct((1, DN), F32),
        ],
    )(x, partials, wnx, wna, wnu, bn1, u, wn2, bn2)


def _glob_body(cs_ref, u_ref, wgu_ref, wgm_ref, bg1_ref, wg2_ref, bg2_ref,
               go_ref):
    mean = cs_ref[...] * (1.0 / N_NODES)
    h = _dot(u_ref[...], wgu_ref[...]) + _dot(mean, wgm_ref[...]) + bg1_ref[...]
    h = jnp.maximum(h, 0.0)
    go_ref[...] = _dot(h, wg2_ref[...]) + bg2_ref[...]


def _global_mlp(colsum, u, wgu, wgm, bg1, wg2, bg2):
    return pl.pallas_call(
        _glob_body,
        out_shape=jax.ShapeDtypeStruct((1, DG), F32),
    )(colsum, u, wgu, wgm, bg1, wg2, bg2)


def kernel(x, edge_index, edge_attr, global_attr,
           We1, be1, We2, be2,
           Wn1, bn1, Wn2, bn2,
           Wg1, bg1, Wg2, bg2):
    row = edge_index[0]
    col = edge_index[1]
    row3 = row.reshape(NW, NCHUNK, CHUNK)
    col3 = col.reshape(NW, NCHUNK, CHUNK)

    w1s = We1[:DN]
    w1d = We1[DN:2 * DN]
    w1e = We1[2 * DN:2 * DN + DE]
    w1u = We1[2 * DN + DE:]
    wnx = Wn1[:DN]
    wna = Wn1[DN:DN + DE]
    wnu = Wn1[DN + DE:]
    wgu = Wg1[:DG]
    wgm = Wg1[DG:]

    be1r = be1.reshape(1, DN)
    be2r = be2.reshape(1, DE)
    bn1r = bn1.reshape(1, DN)
    bn2r = bn2.reshape(1, DN)
    bg1r = bg1.reshape(1, DN)
    bg2r = bg2.reshape(1, DG)

    a_tab, b_tab = _make_tables(x, w1s, w1d, w1u, be1r, global_attr)
    ga, gb = _gather_rows(a_tab, b_tab, row3, col3)
    edge_out = _edge_mlp(ga, gb, edge_attr, w1e, We2, be2r)
    eo4 = edge_out.reshape(NW, NCHUNK, CHUNK, DE)
    zeros2d = jnp.zeros((N_PAD, DE), F32)
    partials = _segment_sum(eo4, col3, zeros2d)
    x_out, colsum = _node_mlp(x, partials, wnx, wna, wnu, bn1r,
                              global_attr, Wn2, bn2r)
    global_out = _global_mlp(colsum, global_attr, wgu, wgm, bg1r, Wg2, bg2r)
    return (x_out, edge_out, global_out)
